# baseline (speedup 1.0000x reference)
import functools

import jax
import jax.numpy as jnp
from jax import lax
from jax.experimental import pallas as pl
from jax.experimental.pallas import tpu as pltpu
from jax.experimental.pallas import tpu_sc as plsc

_N = 100000
_E = 1600000
_V = 72
_TAB = _V * _V
_NC, _NS = 2, 16
_NW = _NC * _NS
_EPW = _E // _NW
_CH = 2000
_NCH = _EPW // _CH
_LANES = 16


def _adj_body(src_hbm, dst_hbm, inv1_hbm, out_hbm, sidx, didx, inv1_v, tab):
    wid = lax.axis_index("s") * _NC + lax.axis_index("c")

    pltpu.sync_copy(inv1_hbm, inv1_v)
    zero = jnp.zeros((_LANES,), jnp.float32)

    def zbody(i, carry):
        tab[pl.ds(i * _LANES, _LANES)] = zero
        return carry

    lax.fori_loop(0, _TAB // _LANES, zbody, 0)

    one = jnp.ones((_LANES,), jnp.float32)

    def chunk(c, carry):
        base = wid * _EPW + c * _CH
        pltpu.sync_copy(src_hbm.at[pl.ds(base, _CH)], sidx)
        pltpu.sync_copy(dst_hbm.at[pl.ds(base, _CH)], didx)

        def group(g, inner):
            s16 = sidx[pl.ds(g * _LANES, _LANES)]
            d16 = didx[pl.ds(g * _LANES, _LANES)]
            es = plsc.load_gather(inv1_v, [s16])
            ed = plsc.load_gather(inv1_v, [d16])
            flat = es * _V + ed
            plsc.store_scatter(tab, [flat], one, mask=es != ed)
            return inner

        return lax.fori_loop(0, _CH // _LANES, group, carry)

    lax.fori_loop(0, _NCH, chunk, 0)
    pltpu.sync_copy(tab, out_hbm.at[wid])


@functools.partial(
    pl.kernel,
    mesh=plsc.VectorSubcoreMesh(core_axis_name="c", subcore_axis_name="s"),
    out_type=jax.ShapeDtypeStruct((_NW, _TAB), jnp.float32),
    compiler_params=pltpu.CompilerParams(needs_layout_passes=False),
    scratch_types=[
        pltpu.VMEM((_CH,), jnp.int32),
        pltpu.VMEM((_CH,), jnp.int32),
        pltpu.VMEM((_N,), jnp.int32),
        pltpu.VMEM((_TAB,), jnp.float32),
    ],
)
def _adj_kernel(src_hbm, dst_hbm, inv1_hbm, out_hbm, sidx, didx, inv1_v, tab):
    _adj_body(src_hbm, dst_hbm, inv1_hbm, out_hbm, sidx, didx, inv1_v, tab)


def _fc_body(hp_ref, w_ref, o_ref):
    o_ref[...] = jnp.dot(hp_ref[...], w_ref[...],
                         preferred_element_type=jnp.float32)


def _fc(hp_flat, fcW):
    return pl.pallas_call(
        _fc_body,
        out_shape=jax.ShapeDtypeStruct((1, 2), jnp.float32),
    )(hp_flat, fcW)


def _seg_max0(data, ids, num):
    out = jax.ops.segment_max(data, ids, num_segments=num)
    return jnp.where(jnp.isneginf(out), 0.0, out)


def kernel(x, pos, edge_index, batch, params):
    src, dst = edge_index[0], edge_index[1]
    N = x.shape[0]

    ix = jnp.floor(pos[:, 0] / 16.0).astype(jnp.int32)
    iy = jnp.floor(pos[:, 1] / 12.0).astype(jnp.int32)
    vox = ix * 9 + iy
    V = _V
    occ = jax.ops.segment_sum(jnp.ones((N,), jnp.float32), vox, num_segments=V) > 0
    rank = jnp.cumsum(occ.astype(jnp.int32)) - 1
    inv1 = rank[vox].astype(jnp.int32)
    K1 = jnp.sum(occ.astype(jnp.int32))
    row_mask = jnp.arange(V, dtype=jnp.int32) < K1
    cnt = jax.ops.segment_sum(jnp.ones((N,), jnp.float32), inv1, num_segments=V)
    cnt_safe = jnp.where(cnt > 0, cnt, 1.0)
    pos_p = jax.ops.segment_sum(pos, inv1, num_segments=V) / cnt_safe[:, None]
    c2 = (jnp.clip(jnp.floor(pos_p[:, 0] / 30.0), 0, 3) * 4
          + jnp.clip(jnp.floor(pos_p[:, 1] / 25.0), 0, 3)).astype(jnp.int32)
    c2 = jnp.where(row_mask, c2, 0)

    adj_parts = _adj_kernel(src, dst, inv1)
    A = (jnp.max(adj_parts, axis=0) > 0.0).reshape(V, V)

    p = params

    def conv(h, k, s, d, n):
        g = h @ p[f"Wf{k}"]
        sm = jax.ops.segment_max(g[s], d, num_segments=n)
        delta = h @ p[f"Wh{k}"] + p[f"bh{k}"]
        aggr = jnp.where(jnp.isneginf(sm), 0.0, sm + delta + p[f"bf{k}"])
        return h @ p[f"Wg{k}"] + p[f"bg{k}"] + aggr

    def conv_pooled(h, k):
        g = h @ p[f"Wf{k}"]
        sm = jnp.max(jnp.where(A[:, :, None], g[:, None, :], -jnp.inf), axis=0)
        delta = h @ p[f"Wh{k}"] + p[f"bh{k}"]
        aggr = jnp.where(jnp.isneginf(sm), 0.0, sm + delta + p[f"bf{k}"])
        return h @ p[f"Wg{k}"] + p[f"bg{k}"] + aggr

    def bn(h, k, rmask=None, count=None):
        if rmask is None:
            mu = h.mean(axis=0)
            var = h.var(axis=0)
        else:
            w = rmask.astype(h.dtype)[:, None]
            mu = jnp.sum(h * w, axis=0) / count
            var = jnp.sum(((h - mu) ** 2) * w, axis=0) / count
        return (h - mu) / jnp.sqrt(var + 1e-5) * p[f"gamma{k}"] + p[f"beta{k}"]

    n = N
    xs = x[:, 0][src]
    mx = jax.ops.segment_max(xs, dst, num_segments=n)
    mn = jax.ops.segment_min(xs, dst, num_segments=n)
    w1 = p["Wf1"][0]
    sm1 = jnp.where(w1 >= 0, mx[:, None] * w1, mn[:, None] * w1)
    delta1 = x @ p["Wh1"] + p["bh1"]
    aggr1 = jnp.where(jnp.isneginf(mx)[:, None], 0.0, sm1 + delta1 + p["bf1"])
    h = x @ p["Wg1"] + p["bg1"] + aggr1
    h = bn(jax.nn.elu(h), 1)
    h = bn(jax.nn.elu(conv(h, 2, src, dst, n)), 2)
    sc = h
    h = bn(jax.nn.elu(conv(h, 3, src, dst, n)), 3)
    h = bn(jax.nn.elu(conv(h, 4, src, dst, n)), 4)
    h = h + sc
    h = bn(jax.nn.elu(conv(h, 5, src, dst, n)), 5)
    K1f = K1.astype(jnp.float32)
    hp = _seg_max0(h, inv1, V)
    sc = hp
    hp = bn(jax.nn.elu(conv_pooled(hp, 6)), 6, row_mask, K1f)
    hp = bn(jax.nn.elu(conv_pooled(hp, 7)), 7, row_mask, K1f)
    hp = hp + sc
    hp = jnp.where(row_mask[:, None], hp, -jnp.inf)
    out16 = _seg_max0(hp, c2, 16)
    return _fc(out16.reshape(1, 32 * 16), p["fcW"])


# speedup vs baseline: 1.1962x; 1.1962x over previous
import functools

import jax
import jax.numpy as jnp
from jax import lax
from jax.experimental import pallas as pl
from jax.experimental.pallas import tpu as pltpu
from jax.experimental.pallas import tpu_sc as plsc

_N = 100000
_E = 1600000
_V = 72
_TAB = _V * _V
_NC, _NS = 2, 16
_NW = _NC * _NS
_EPW = _E // _NW
_CH = 2000
_NCH = _EPW // _CH
_LANES = 16


def _adj_body(src_hbm, dst_hbm, inv1_hbm, out_hbm, sidx, didx, inv1_v, tab):
    wid = lax.axis_index("s") * _NC + lax.axis_index("c")

    pltpu.sync_copy(inv1_hbm, inv1_v)
    zero = jnp.zeros((_LANES,), jnp.float32)

    def zbody(i, carry):
        tab[pl.ds(i * _LANES, _LANES)] = zero
        return carry

    lax.fori_loop(0, _TAB // _LANES, zbody, 0)

    one = jnp.ones((_LANES,), jnp.float32)

    def chunk(c, carry):
        base = wid * _EPW + c * _CH
        pltpu.sync_copy(src_hbm.at[pl.ds(base, _CH)], sidx)
        pltpu.sync_copy(dst_hbm.at[pl.ds(base, _CH)], didx)

        def group(g, inner):
            s16 = sidx[pl.ds(g * _LANES, _LANES)]
            d16 = didx[pl.ds(g * _LANES, _LANES)]
            es = plsc.load_gather(inv1_v, [s16])
            ed = plsc.load_gather(inv1_v, [d16])
            flat = es * _V + ed
            plsc.store_scatter(tab, [flat], one, mask=es != ed)
            return inner

        return lax.fori_loop(0, _CH // _LANES, group, carry)

    lax.fori_loop(0, _NCH, chunk, 0)
    pltpu.sync_copy(tab, out_hbm.at[wid])


@functools.partial(
    pl.kernel,
    mesh=plsc.VectorSubcoreMesh(core_axis_name="c", subcore_axis_name="s"),
    out_type=jax.ShapeDtypeStruct((_NW, _TAB), jnp.float32),
    compiler_params=pltpu.CompilerParams(needs_layout_passes=False),
    scratch_types=[
        pltpu.VMEM((_CH,), jnp.int32),
        pltpu.VMEM((_CH,), jnp.int32),
        pltpu.VMEM((_N,), jnp.int32),
        pltpu.VMEM((_TAB,), jnp.float32),
    ],
)
def _adj_kernel(src_hbm, dst_hbm, inv1_hbm, out_hbm, sidx, didx, inv1_v, tab):
    _adj_body(src_hbm, dst_hbm, inv1_hbm, out_hbm, sidx, didx, inv1_v, tab)


def _fc_body(hp_ref, w_ref, o_ref):
    o_ref[...] = jnp.dot(hp_ref[...], w_ref[...],
                         preferred_element_type=jnp.float32)


def _fc(hp_flat, fcW):
    return pl.pallas_call(
        _fc_body,
        out_shape=jax.ShapeDtypeStruct((1, 2), jnp.float32),
    )(hp_flat, fcW)


def _seg_max0(data, ids, num):
    out = jax.ops.segment_max(data, ids, num_segments=num)
    return jnp.where(jnp.isneginf(out), 0.0, out)


def kernel(x, pos, edge_index, batch, params):
    src, dst = edge_index[0], edge_index[1]
    N = x.shape[0]

    ix = jnp.floor(pos[:, 0] / 16.0).astype(jnp.int32)
    iy = jnp.floor(pos[:, 1] / 12.0).astype(jnp.int32)
    vox = ix * 9 + iy
    V = _V
    occ = jax.ops.segment_sum(jnp.ones((N,), jnp.float32), vox, num_segments=V) > 0
    rank = jnp.cumsum(occ.astype(jnp.int32)) - 1
    inv1 = rank[vox].astype(jnp.int32)
    K1 = jnp.sum(occ.astype(jnp.int32))
    row_mask = jnp.arange(V, dtype=jnp.int32) < K1
    cnt = jax.ops.segment_sum(jnp.ones((N,), jnp.float32), inv1, num_segments=V)
    cnt_safe = jnp.where(cnt > 0, cnt, 1.0)
    pos_p = jax.ops.segment_sum(pos, inv1, num_segments=V) / cnt_safe[:, None]
    c2 = (jnp.clip(jnp.floor(pos_p[:, 0] / 30.0), 0, 3) * 4
          + jnp.clip(jnp.floor(pos_p[:, 1] / 25.0), 0, 3)).astype(jnp.int32)
    c2 = jnp.where(row_mask, c2, 0)

    adj_parts = _adj_kernel(src, dst, inv1)
    A = (jnp.max(adj_parts, axis=0) > 0.0).reshape(V, V)

    p = params

    def conv(h, k, s, d, n):
        g = h @ p[f"Wf{k}"]
        sm = jax.ops.segment_max(g[s], d, num_segments=n)
        delta = h @ p[f"Wh{k}"] + p[f"bh{k}"]
        aggr = jnp.where(jnp.isneginf(sm), 0.0, sm + delta + p[f"bf{k}"])
        return h @ p[f"Wg{k}"] + p[f"bg{k}"] + aggr

    def conv_pooled(h, k):
        g = h @ p[f"Wf{k}"]
        sm = jnp.max(jnp.where(A[:, :, None], g[:, None, :], -jnp.inf), axis=0)
        delta = h @ p[f"Wh{k}"] + p[f"bh{k}"]
        aggr = jnp.where(jnp.isneginf(sm), 0.0, sm + delta + p[f"bf{k}"])
        return h @ p[f"Wg{k}"] + p[f"bg{k}"] + aggr

    def bn(h, k, rmask=None, count=None):
        if rmask is None:
            mu = h.mean(axis=0)
            var = h.var(axis=0)
        else:
            w = rmask.astype(h.dtype)[:, None]
            mu = jnp.sum(h * w, axis=0) / count
            var = jnp.sum(((h - mu) ** 2) * w, axis=0) / count
        return (h - mu) / jnp.sqrt(var + 1e-5) * p[f"gamma{k}"] + p[f"beta{k}"]

    n = N
    h = bn(jax.nn.elu(conv(x, 1, src, dst, n)), 1)
    h = bn(jax.nn.elu(conv(h, 2, src, dst, n)), 2)
    sc = h
    h = bn(jax.nn.elu(conv(h, 3, src, dst, n)), 3)
    h = bn(jax.nn.elu(conv(h, 4, src, dst, n)), 4)
    h = h + sc
    h = bn(jax.nn.elu(conv(h, 5, src, dst, n)), 5)
    K1f = K1.astype(jnp.float32)
    hp = _seg_max0(h, inv1, V)
    sc = hp
    hp = bn(jax.nn.elu(conv_pooled(hp, 6)), 6, row_mask, K1f)
    hp = bn(jax.nn.elu(conv_pooled(hp, 7)), 7, row_mask, K1f)
    hp = hp + sc
    hp = jnp.where(row_mask[:, None], hp, -jnp.inf)
    out16 = _seg_max0(hp, c2, 16)
    return _fc(out16.reshape(1, 32 * 16), p["fcW"])
